# baseline (speedup 1.0000x reference)
import functools

import jax
import jax.numpy as jnp
from jax import lax
from jax.experimental import pallas as pl
from jax.experimental.pallas import tpu as pltpu
from jax.experimental.pallas import tpu_sc as plsc

N = 100000
D = 128
G = 128
NC, NS = 2, 16
NW = NC * NS
GPW = 25
RPW = GPW * G
NFULL = N // G
TAIL = N - NFULL * G
LASTW = NW - 1
LASTW_ROWS = N - LASTW * RPW

_mesh = plsc.VectorSubcoreMesh(core_axis_name="c", subcore_axis_name="s")


@functools.partial(
    pl.kernel,
    mesh=_mesh,
    out_type=jax.ShapeDtypeStruct((N, D), jnp.float32),
    scratch_types=[
        pltpu.VMEM((RPW,), jnp.int32),
        pltpu.VMEM((3, G, D), jnp.float32),
        pltpu.VMEM_SHARED((28, D), jnp.float32),
        pltpu.SemaphoreType.DMA,
        pltpu.SemaphoreType.DMA,
    ],
)
def _emb_lookup(idx_hbm, table_hbm, out_hbm, idx_v, rows_v, table_v,
                gsem, ssem):
    sid = lax.axis_index("s")
    wid = sid * NC + lax.axis_index("c")

    @pl.when(sid == 0)
    def _stage_table():
        pltpu.sync_copy(table_hbm, table_v)

    plsc.subcore_barrier()

    base = pl.multiple_of(wid * RPW, G)

    @pl.when(wid < LASTW)
    def _stage_idx():
        pltpu.sync_copy(idx_hbm.at[pl.ds(base, RPW)], idx_v)

    @pl.when(wid == LASTW)
    def _stage_idx_last():
        pltpu.sync_copy(idx_hbm.at[pl.ds(base, LASTW_ROWS)],
                        idx_v.at[pl.ds(0, LASTW_ROWS)])

    def fire_gather(g, pb):
        goff = pl.multiple_of(g * G, G)
        chunk = wid * GPW + g

        @pl.when(chunk < NFULL)
        def _():
            pltpu.async_copy(table_v.at[idx_v.at[pl.ds(goff, G)]],
                             rows_v.at[pb], gsem)

        @pl.when(chunk == NFULL)
        def _():
            pltpu.async_copy(table_v.at[idx_v.at[pl.ds(goff, TAIL)]],
                             rows_v.at[pb, pl.ds(0, TAIL)], gsem)

    def drain_gather(g, pb):
        chunk = wid * GPW + g

        @pl.when(chunk < NFULL)
        def _():
            pltpu.make_async_copy(out_hbm.at[pl.ds(0, G)], rows_v.at[pb], gsem).wait()

        @pl.when(chunk == NFULL)
        def _():
            pltpu.make_async_copy(out_hbm.at[pl.ds(0, TAIL)],
                                  rows_v.at[pb, pl.ds(0, TAIL)], gsem).wait()

    def fire_store(g, pb):
        chunk = wid * GPW + g
        row0 = pl.multiple_of(chunk * G, G)

        @pl.when(chunk < NFULL)
        def _():
            pltpu.async_copy(rows_v.at[pb], out_hbm.at[pl.ds(row0, G)], ssem)

        @pl.when(chunk == NFULL)
        def _():
            pltpu.async_copy(rows_v.at[pb, pl.ds(0, TAIL)],
                             out_hbm.at[pl.ds(row0, TAIL)], ssem)

    def drain_store(g, pb):
        chunk = wid * GPW + g

        @pl.when(chunk < NFULL)
        def _():
            pltpu.make_async_copy(out_hbm.at[pl.ds(0, G)], rows_v.at[pb], ssem).wait()

        @pl.when(chunk == NFULL)
        def _():
            pltpu.make_async_copy(out_hbm.at[pl.ds(0, TAIL)],
                                  rows_v.at[pb, pl.ds(0, TAIL)], ssem).wait()

    fire_gather(0, 0)

    def body(g, _):
        pb = lax.rem(g, 3)
        pn = lax.rem(g + 1, 3)

        drain_gather(g, pb)

        @pl.when(g >= 2)
        def _():
            drain_store(g - 2, pn)

        @pl.when(g + 1 < GPW)
        def _():
            fire_gather(g + 1, pn)

        fire_store(g, pb)
        return ()

    lax.fori_loop(0, GPW, body, ())

    drain_store(GPW - 2, (GPW - 2) % 3)
    drain_store(GPW - 1, (GPW - 1) % 3)


def kernel(x, table):
    return _emb_lookup(x[:, 0].astype(jnp.int32), table)


# speedup vs baseline: 1.0077x; 1.0077x over previous
import functools

import jax
import jax.numpy as jnp
from jax import lax
from jax.experimental import pallas as pl
from jax.experimental.pallas import tpu as pltpu
from jax.experimental.pallas import tpu_sc as plsc

N = 100000
D = 128
G = 80
GPS = 4
SC_ROWS = G * GPS
NC, NS = 2, 16
NW = NC * NS
SPW = 10
RPW = SPW * SC_ROWS
NFULL = N // SC_ROWS
TAIL = N - NFULL * SC_ROWS
TGATH = TAIL // G
LASTW = NW - 1
LASTW_ROWS = N - LASTW * RPW

_mesh = plsc.VectorSubcoreMesh(core_axis_name="c", subcore_axis_name="s")


@functools.partial(
    pl.kernel,
    mesh=_mesh,
    out_type=jax.ShapeDtypeStruct((N, D), jnp.float32),
    scratch_types=[
        pltpu.VMEM((RPW,), jnp.int32),
        pltpu.VMEM((2, SC_ROWS, D), jnp.float32),
        pltpu.VMEM_SHARED((28, D), jnp.float32),
        pltpu.SemaphoreType.DMA,
        pltpu.SemaphoreType.DMA,
    ],
)
def _emb_lookup(idx_hbm, table_hbm, out_hbm, idx_v, rows_v, table_v,
                gsem, ssem):
    sid = lax.axis_index("s")
    wid = sid * NC + lax.axis_index("c")

    @pl.when(sid == 0)
    def _stage_table():
        pltpu.sync_copy(table_hbm, table_v)

    plsc.subcore_barrier()

    base = pl.multiple_of(wid * RPW, 8)

    @pl.when(wid < LASTW)
    def _stage_idx():
        pltpu.sync_copy(idx_hbm.at[pl.ds(base, RPW)], idx_v)

    @pl.when(wid == LASTW)
    def _stage_idx_last():
        pltpu.sync_copy(idx_hbm.at[pl.ds(base, LASTW_ROWS)],
                        idx_v.at[pl.ds(0, LASTW_ROWS)])

    def fire_gathers(t, pb):
        chunk = wid * SPW + t

        @pl.when(chunk < NFULL)
        def _():
            for j in range(GPS):
                goff = pl.multiple_of(t * SC_ROWS + j * G, 8)
                pltpu.async_copy(table_v.at[idx_v.at[pl.ds(goff, G)]],
                                 rows_v.at[pb, pl.ds(j * G, G)], gsem)

        @pl.when(chunk == NFULL)
        def _():
            for j in range(TGATH):
                goff = pl.multiple_of(t * SC_ROWS + j * G, 8)
                pltpu.async_copy(table_v.at[idx_v.at[pl.ds(goff, G)]],
                                 rows_v.at[pb, pl.ds(j * G, G)], gsem)

    def drain_gathers(t, pb):
        chunk = wid * SPW + t

        @pl.when(chunk < NFULL)
        def _():
            pltpu.make_async_copy(out_hbm.at[pl.ds(0, SC_ROWS)],
                                  rows_v.at[pb], gsem).wait()

        @pl.when(chunk == NFULL)
        def _():
            pltpu.make_async_copy(out_hbm.at[pl.ds(0, TAIL)],
                                  rows_v.at[pb, pl.ds(0, TAIL)], gsem).wait()

    def fire_store(t, pb):
        chunk = wid * SPW + t
        row0 = pl.multiple_of(chunk * SC_ROWS, 8)

        @pl.when(chunk < NFULL)
        def _():
            pltpu.async_copy(rows_v.at[pb],
                             out_hbm.at[pl.ds(row0, SC_ROWS)], ssem)

        @pl.when(chunk == NFULL)
        def _():
            pltpu.async_copy(rows_v.at[pb, pl.ds(0, TAIL)],
                             out_hbm.at[pl.ds(row0, TAIL)], ssem)

    def drain_store(t, pb):
        chunk = wid * SPW + t

        @pl.when(chunk < NFULL)
        def _():
            pltpu.make_async_copy(out_hbm.at[pl.ds(0, SC_ROWS)],
                                  rows_v.at[pb], ssem).wait()

        @pl.when(chunk == NFULL)
        def _():
            pltpu.make_async_copy(out_hbm.at[pl.ds(0, TAIL)],
                                  rows_v.at[pb, pl.ds(0, TAIL)], ssem).wait()

    fire_gathers(0, 0)

    def body(t, _):
        pb = lax.rem(t, 2)
        pn = 1 - pb

        drain_gathers(t, pb)

        @pl.when(t >= 1)
        def _():
            drain_store(t - 1, pn)

        @pl.when(t + 1 < SPW)
        def _():
            fire_gathers(t + 1, pn)

        fire_store(t, pb)
        return ()

    lax.fori_loop(0, SPW, body, ())

    drain_store(SPW - 1, (SPW - 1) % 2)


def kernel(x, table):
    return _emb_lookup(x[:, 0].astype(jnp.int32), table)


# speedup vs baseline: 1.0125x; 1.0047x over previous
import functools

import jax
import jax.numpy as jnp
from jax import lax
from jax.experimental import pallas as pl
from jax.experimental.pallas import tpu as pltpu
from jax.experimental.pallas import tpu_sc as plsc

N = 100000
D = 128
G = 112
GPS = 4
SC_ROWS = G * GPS
NC, NS = 2, 16
NW = NC * NS
SPW = 7
RPW = SPW * SC_ROWS
NFULL = N // SC_ROWS
TAIL = N - NFULL * SC_ROWS
LASTW = NW - 1
LASTW_ROWS = N - LASTW * RPW

_mesh = plsc.VectorSubcoreMesh(core_axis_name="c", subcore_axis_name="s")


@functools.partial(
    pl.kernel,
    mesh=_mesh,
    out_type=jax.ShapeDtypeStruct((N, D), jnp.float32),
    scratch_types=[
        pltpu.VMEM((RPW,), jnp.int32),
        pltpu.VMEM((2, SC_ROWS, D), jnp.float32),
        pltpu.VMEM_SHARED((28, D), jnp.float32),
        pltpu.SemaphoreType.DMA,
        pltpu.SemaphoreType.DMA,
    ],
)
def _emb_lookup(idx_hbm, table_hbm, out_hbm, idx_v, rows_v, table_v,
                gsem, ssem):
    sid = lax.axis_index("s")
    wid = sid * NC + lax.axis_index("c")

    @pl.when(sid == 0)
    def _stage_table():
        pltpu.sync_copy(table_hbm, table_v)

    plsc.subcore_barrier()

    base = pl.multiple_of(wid * RPW, 8)

    @pl.when(wid < LASTW)
    def _stage_idx():
        pltpu.sync_copy(idx_hbm.at[pl.ds(base, RPW)], idx_v)

    @pl.when(wid == LASTW)
    def _stage_idx_last():
        pltpu.sync_copy(idx_hbm.at[pl.ds(base, LASTW_ROWS)],
                        idx_v.at[pl.ds(0, LASTW_ROWS)])

    def fire_gathers(t, pb):
        chunk = wid * SPW + t

        @pl.when(chunk < NFULL)
        def _():
            for j in range(GPS):
                goff = pl.multiple_of(t * SC_ROWS + j * G, 8)
                pltpu.async_copy(table_v.at[idx_v.at[pl.ds(goff, G)]],
                                 rows_v.at[pb, pl.ds(j * G, G)], gsem)

        @pl.when(chunk == NFULL)
        def _():
            goff = pl.multiple_of(t * SC_ROWS, 8)
            pltpu.async_copy(table_v.at[idx_v.at[pl.ds(goff, TAIL)]],
                             rows_v.at[pb, pl.ds(0, TAIL)], gsem)

    def drain_gathers(t, pb):
        chunk = wid * SPW + t

        @pl.when(chunk < NFULL)
        def _():
            pltpu.make_async_copy(out_hbm.at[pl.ds(0, SC_ROWS)],
                                  rows_v.at[pb], gsem).wait()

        @pl.when(chunk == NFULL)
        def _():
            pltpu.make_async_copy(out_hbm.at[pl.ds(0, TAIL)],
                                  rows_v.at[pb, pl.ds(0, TAIL)], gsem).wait()

    def fire_store(t, pb):
        chunk = wid * SPW + t
        row0 = pl.multiple_of(chunk * SC_ROWS, 8)

        @pl.when(chunk < NFULL)
        def _():
            pltpu.async_copy(rows_v.at[pb],
                             out_hbm.at[pl.ds(row0, SC_ROWS)], ssem)

        @pl.when(chunk == NFULL)
        def _():
            pltpu.async_copy(rows_v.at[pb, pl.ds(0, TAIL)],
                             out_hbm.at[pl.ds(row0, TAIL)], ssem)

    def drain_store(t, pb):
        chunk = wid * SPW + t

        @pl.when(chunk < NFULL)
        def _():
            pltpu.make_async_copy(out_hbm.at[pl.ds(0, SC_ROWS)],
                                  rows_v.at[pb], ssem).wait()

        @pl.when(chunk == NFULL)
        def _():
            pltpu.make_async_copy(out_hbm.at[pl.ds(0, TAIL)],
                                  rows_v.at[pb, pl.ds(0, TAIL)], ssem).wait()

    fire_gathers(0, 0)

    def body(t, _):
        pb = lax.rem(t, 2)
        pn = 1 - pb

        drain_gathers(t, pb)

        @pl.when(t >= 1)
        def _():
            drain_store(t - 1, pn)

        @pl.when(t + 1 < SPW)
        def _():
            fire_gathers(t + 1, pn)

        fire_store(t, pb)
        return ()

    lax.fori_loop(0, SPW, body, ())

    drain_store(SPW - 1, (SPW - 1) % 2)


def kernel(x, table):
    return _emb_lookup(x[:, 0].astype(jnp.int32), table)
